# baseline (speedup 1.0000x reference)
import functools

import jax
import jax.numpy as jnp
from jax import lax
from jax.experimental import pallas as pl
from jax.experimental.pallas import tpu as pltpu
from jax.experimental.pallas import tpu_sc as plsc

D = 64
B_TOTAL = 4096 * 200
NW = 32
B_PER_W = B_TOTAL // NW
ROW = 128
K = 8
CHUNK = K * ROW
N_CHUNKS = B_PER_W // CHUNK
IDX_ROWS_PER_W = B_PER_W // ROW

_mesh = plsc.VectorSubcoreMesh(core_axis_name="c", subcore_axis_name="s")


@functools.partial(
    pl.kernel,
    mesh=_mesh,
    out_type=jax.ShapeDtypeStruct((B_TOTAL, D), jnp.float32),
    scratch_types=[
        pltpu.VMEM((K, ROW), jnp.int32),
        pltpu.VMEM((CHUNK, D), jnp.float32),
        pltpu.SemaphoreType.DMA,
    ],
    compiler_params=pltpu.CompilerParams(use_tc_tiling_on_sc=False),
)
def _gather_kernel(table_hbm, idx_hbm, out_hbm, idx_v, rows_v, sem):
    wid = lax.axis_index("s") * 2 + lax.axis_index("c")
    idx_row0 = wid * IDX_ROWS_PER_W
    out_base = wid * B_PER_W

    def body(i, _):
        pltpu.sync_copy(idx_hbm.at[pl.ds(idx_row0 + i * K, K)], idx_v)
        copies = [
            pltpu.async_copy(
                table_hbm.at[idx_v.at[j]],
                rows_v.at[pl.ds(j * ROW, ROW)],
                sem,
            )
            for j in range(K)
        ]
        for c in copies:
            c.wait()
        pltpu.sync_copy(rows_v, out_hbm.at[pl.ds(out_base + i * CHUNK, CHUNK)])
        return 0

    lax.fori_loop(0, N_CHUNKS, body, 0)


def kernel(idx, embed_weight):
    idx2d = idx.astype(jnp.int32).reshape(-1, ROW)
    out = _gather_kernel(embed_weight, idx2d)
    return out.reshape(idx.shape + (D,))


# speedup vs baseline: 1.0106x; 1.0106x over previous
import functools

import jax
import jax.numpy as jnp
from jax import lax
from jax.experimental import pallas as pl
from jax.experimental.pallas import tpu as pltpu
from jax.experimental.pallas import tpu_sc as plsc

D = 64
B_TOTAL = 4096 * 200
NW = 32
B_PER_W = B_TOTAL // NW
ROW = 128
K = 5
CHUNK = K * ROW
N_CHUNKS = B_PER_W // CHUNK
IDX_ROWS_PER_W = B_PER_W // ROW

_mesh = plsc.VectorSubcoreMesh(core_axis_name="c", subcore_axis_name="s")


@functools.partial(
    pl.kernel,
    mesh=_mesh,
    out_type=jax.ShapeDtypeStruct((B_TOTAL, D), jnp.float32),
    scratch_types=[
        pltpu.VMEM((K, ROW), jnp.int32),
        pltpu.VMEM((K, ROW), jnp.int32),
        pltpu.VMEM((CHUNK, D), jnp.float32),
        pltpu.VMEM((CHUNK, D), jnp.float32),
        pltpu.SemaphoreType.DMA,
        pltpu.SemaphoreType.DMA,
    ],
    compiler_params=pltpu.CompilerParams(use_tc_tiling_on_sc=False),
)
def _gather_kernel(table_hbm, idx_hbm, out_hbm, idx0, idx1, rows0, rows1,
                   gsem0, gsem1):
    wid = lax.axis_index("s") * 2 + lax.axis_index("c")
    idx_row0 = wid * IDX_ROWS_PER_W
    out_base = wid * B_PER_W

    def fire(i, idx_buf, row_buf, sem):
        pltpu.sync_copy(idx_hbm.at[pl.ds(idx_row0 + i * K, K)], idx_buf)
        for j in range(K):
            pltpu.async_copy(
                table_hbm.at[idx_buf.at[j]],
                row_buf.at[pl.ds(j * ROW, ROW)],
                sem,
            )

    def drain(row_buf, sem):
        pltpu.make_async_copy(table_hbm.at[pl.ds(0, CHUNK)], row_buf, sem).wait()

    def writeback(row_buf, i):
        pltpu.sync_copy(row_buf, out_hbm.at[pl.ds(out_base + i * CHUNK, CHUNK)])

    fire(0, idx0, rows0, gsem0)

    def body(t, _):
        a = 2 * t
        fire(a + 1, idx1, rows1, gsem1)
        drain(rows0, gsem0)
        writeback(rows0, a)

        @pl.when(a + 2 < N_CHUNKS)
        def _():
            fire(a + 2, idx0, rows0, gsem0)

        drain(rows1, gsem1)
        writeback(rows1, a + 1)
        return 0

    lax.fori_loop(0, N_CHUNKS // 2, body, 0)


def kernel(idx, embed_weight):
    idx2d = idx.astype(jnp.int32).reshape(-1, ROW)
    out = _gather_kernel(embed_weight, idx2d)
    return out.reshape(idx.shape + (D,))
